# baseline (speedup 1.0000x reference)
import dataclasses
import functools

import jax
import jax.numpy as jnp
from jax import lax
from jax.experimental import pallas as pl
from jax.experimental.pallas import tpu as pltpu
from jax.experimental.pallas import tpu_sc as plsc

E = 16
L = 16
D = 2048
B = 4
H = 16
S = 4096
DH = D // H


def _sc_gather(ids16, gates, ep_rows):
    mesh = plsc.VectorSubcoreMesh(core_axis_name="c", subcore_axis_name="s")
    cp = pltpu.CompilerParams()
    if "needs_layout_passes" in pltpu.CompilerParams.__dataclass_fields__:
        cp = dataclasses.replace(cp, needs_layout_passes=False)

    @functools.partial(
        pl.kernel,
        out_type=(
            jax.ShapeDtypeStruct((B * L, D), jnp.float32),
            jax.ShapeDtypeStruct((E,), jnp.float32),
        ),
        mesh=mesh,
        compiler_params=cp,
        scratch_types=[
            pltpu.VMEM((16,), jnp.int32),
            pltpu.VMEM((16,), jnp.int32),
            pltpu.VMEM((L, D), jnp.float32),
            pltpu.VMEM((16,), jnp.float32),
            pltpu.SemaphoreType.DMA,
        ],
    )
    def k(ids_hbm, g_hbm, ep_hbm, out_hbm, sg_hbm, tid_v, idx_v, rows_v, g_v, sem):
        wid = lax.axis_index("s") * 2 + lax.axis_index("c")

        @pl.when(wid < B)
        def _():
            pltpu.sync_copy(ids_hbm, tid_v)
            bvec = jnp.full((16,), wid, jnp.int32)
            tid_b = plsc.load_gather(tid_v, [bvec])
            idx_v[...] = tid_b * L + lax.iota(jnp.int32, 16)
            pltpu.async_copy(ep_hbm.at[idx_v], rows_v, sem).wait()
            pltpu.sync_copy(rows_v, out_hbm.at[pl.ds(wid * L, L)])

        @pl.when(wid == B)
        def _():
            pltpu.sync_copy(g_hbm, g_v)
            g_v[...] = 1.0 / (1.0 + jnp.exp(-g_v[...]))
            pltpu.sync_copy(g_v, sg_hbm)

    return k(ids16, gates, ep_rows)


_HB = 4
_HG = H // _HB
_WN = _HB * DH


def _mega_body(tid_ref, sg_ref, x_ref, wk_ref, wv_ref, aw_ref, k_ref, o_ref):
    b = pl.program_id(1)
    g = sg_ref[tid_ref[b]]
    x = x_ref[...]
    k_ref[0] = jnp.dot(x, wk_ref[...], preferred_element_type=jnp.float32)
    v = jnp.dot(x, wv_ref[...], preferred_element_type=jnp.float32)
    for hh in range(_HB):
        xt = aw_ref[0, hh]
        e = jnp.exp(xt)
        z = jnp.sum(e, axis=0, keepdims=True)
        r = (e * (g / z)).astype(jnp.bfloat16)
        vb = v[:, hh * DH:(hh + 1) * DH].astype(jnp.bfloat16)
        o_ref[0, hh] = lax.dot_general(
            r, vb, (((0,), (0,)), ((), ())),
            preferred_element_type=jnp.float32,
        )


def _mega(x2d, Wk, Wv, aw_t, sg, task_ids, interpret=False):
    return pl.pallas_call(
        _mega_body,
        grid=(_HG, B),
        in_specs=[
            pl.BlockSpec(memory_space=pltpu.SMEM),
            pl.BlockSpec(memory_space=pltpu.SMEM),
            pl.BlockSpec((L, D), lambda hg, b: (b, 0)),
            pl.BlockSpec((D, _WN), lambda hg, b: (0, hg)),
            pl.BlockSpec((D, _WN), lambda hg, b: (0, hg)),
            pl.BlockSpec((1, _HB, L, S), lambda hg, b: (b, hg, 0, 0)),
        ],
        out_specs=[
            pl.BlockSpec((1, L, _WN), lambda hg, b: (b, 0, hg)),
            pl.BlockSpec((1, _HB, S, DH), lambda hg, b: (b, hg, 0, 0)),
        ],
        out_shape=[
            jax.ShapeDtypeStruct((B, L, D), jnp.float32),
            jax.ShapeDtypeStruct((B, H, S, DH), jnp.float32),
        ],
        compiler_params=pltpu.CompilerParams(
            dimension_semantics=("parallel", "parallel"),
        ),
        interpret=interpret,
    )(task_ids, sg, x2d, Wk, Wv, aw_t)


def kernel(task_ids, expert_prompts, Wk, Wv, gates, adapter_weights):
    task_ids = task_ids.astype(jnp.int32)
    ids16 = jnp.zeros((16,), jnp.int32).at[:B].set(task_ids)
    x2d, sg = _sc_gather(ids16, gates, expert_prompts.reshape(E * L, D))
    aw_t = jnp.swapaxes(adapter_weights, 2, 3)
    adapter_k, out = _mega(x2d, Wk, Wv, aw_t, sg, task_ids)
    return out, adapter_k


# speedup vs baseline: 1.0553x; 1.0553x over previous
import dataclasses
import functools

import jax
import jax.numpy as jnp
from jax import lax
from jax.experimental import pallas as pl
from jax.experimental.pallas import tpu as pltpu
from jax.experimental.pallas import tpu_sc as plsc

E = 16
L = 16
D = 2048
B = 4
H = 16
S = 4096
DH = D // H


def _sc_gather(ids16, ep_rows):
    mesh = plsc.VectorSubcoreMesh(core_axis_name="c", subcore_axis_name="s")
    cp = pltpu.CompilerParams()
    if "needs_layout_passes" in pltpu.CompilerParams.__dataclass_fields__:
        cp = dataclasses.replace(cp, needs_layout_passes=False)

    @functools.partial(
        pl.kernel,
        out_type=jax.ShapeDtypeStruct((B * L, D), jnp.float32),
        mesh=mesh,
        compiler_params=cp,
        scratch_types=[
            pltpu.VMEM((16,), jnp.int32),
            pltpu.VMEM((16,), jnp.int32),
            pltpu.VMEM((L, D), jnp.float32),
            pltpu.SemaphoreType.DMA,
        ],
    )
    def k(ids_hbm, ep_hbm, out_hbm, tid_v, idx_v, rows_v, sem):
        wid = lax.axis_index("s") * 2 + lax.axis_index("c")

        @pl.when(wid < B)
        def _():
            pltpu.sync_copy(ids_hbm, tid_v)
            bvec = jnp.full((16,), wid, jnp.int32)
            tid_b = plsc.load_gather(tid_v, [bvec])
            idx_v[...] = tid_b * L + lax.iota(jnp.int32, 16)
            pltpu.async_copy(ep_hbm.at[idx_v], rows_v, sem).wait()
            pltpu.sync_copy(rows_v, out_hbm.at[pl.ds(wid * L, L)])

    return k(ids16, ep_rows)


_BK = 256


def _proj_body(x_ref, wk_ref, wv_ref, g_ref, k_ref, v_ref, sg_ref):
    j = pl.program_id(0)
    x = x_ref[...]
    pk = jnp.dot(x, wk_ref[...], preferred_element_type=jnp.float32)
    pv = jnp.dot(x, wv_ref[...], preferred_element_type=jnp.float32)

    @pl.when(j == 0)
    def _():
        k_ref[...] = pk
        v_ref[...] = pv
        sg_ref[...] = jax.nn.sigmoid(g_ref[...])

    @pl.when(j > 0)
    def _():
        k_ref[...] += pk
        v_ref[...] += pv


def _proj(x2d, Wk, Wv, gates2d, interpret=False):
    return pl.pallas_call(
        _proj_body,
        grid=(D // _BK,),
        in_specs=[
            pl.BlockSpec((B * L, _BK), lambda j: (0, j)),
            pl.BlockSpec((_BK, D), lambda j: (j, 0)),
            pl.BlockSpec((_BK, D), lambda j: (j, 0)),
            pl.BlockSpec((1, E), lambda j: (0, 0)),
        ],
        out_specs=[
            pl.BlockSpec((B * L, D), lambda j: (0, 0)),
            pl.BlockSpec((B * L, D), lambda j: (0, 0)),
            pl.BlockSpec((1, E), lambda j: (0, 0)),
        ],
        out_shape=[
            jax.ShapeDtypeStruct((B * L, D), jnp.float32),
            jax.ShapeDtypeStruct((B * L, D), jnp.float32),
            jax.ShapeDtypeStruct((1, E), jnp.float32),
        ],
        interpret=interpret,
    )(x2d, Wk, Wv, gates2d)


_HB = 8


def _attend_body(tid_ref, sg_ref, aw_ref, v_ref, o_ref):
    b = pl.program_id(0)
    g = sg_ref[0, tid_ref[b]]
    for hh in range(_HB):
        xt = aw_ref[0, hh]
        e = jnp.exp(xt)
        z = jnp.sum(e, axis=0, keepdims=True)
        r = (e * (g / z)).astype(jnp.bfloat16)
        vb = v_ref[:, hh * DH:(hh + 1) * DH].astype(jnp.bfloat16)
        o_ref[0, hh] = lax.dot_general(
            r, vb, (((0,), (0,)), ((), ())),
            preferred_element_type=jnp.float32,
        )


def _attend(aw_t, v2d, sg, task_ids, interpret=False):
    return pl.pallas_call(
        _attend_body,
        grid=(B, H // _HB),
        in_specs=[
            pl.BlockSpec(memory_space=pltpu.SMEM),
            pl.BlockSpec(memory_space=pltpu.SMEM),
            pl.BlockSpec((1, _HB, L, S), lambda b, h: (b, h, 0, 0)),
            pl.BlockSpec((L, _HB * DH), lambda b, h: (b, h)),
        ],
        out_specs=pl.BlockSpec((1, _HB, S, DH), lambda b, h: (b, h, 0, 0)),
        out_shape=jax.ShapeDtypeStruct((B, H, S, DH), jnp.float32),
        compiler_params=pltpu.CompilerParams(
            dimension_semantics=("parallel", "parallel"),
        ),
        interpret=interpret,
    )(task_ids, sg, aw_t, v2d)


def kernel(task_ids, expert_prompts, Wk, Wv, gates, adapter_weights):
    task_ids = task_ids.astype(jnp.int32)
    ids16 = jnp.zeros((16,), jnp.int32).at[:B].set(task_ids)
    x2d = _sc_gather(ids16, expert_prompts.reshape(E * L, D))
    k2d, v2d, sg = _proj(x2d, Wk, Wv, gates.reshape(1, E))
    adapter_k = k2d.reshape(B, L, D)
    aw_t = jnp.swapaxes(adapter_weights, 2, 3)
    out = _attend(aw_t, v2d, sg, task_ids)
    return out, adapter_k
